# baseline (speedup 1.0000x reference)
import functools

import jax
import jax.numpy as jnp
import numpy as np
from jax import lax
from jax.experimental import pallas as pl
from jax.experimental.pallas import tpu as pltpu
from jax.experimental.pallas import tpu_sc as plsc

F32 = jnp.float32
I32 = jnp.int32

NC = 2
NS = 16
NW = NC * NS
LANES = 16
CH = 128


def _cdiv(a, b):
  return (a + b - 1) // b


def _sc_segsum_scalar(idx, w, nseg):
  p = _sc_gather_scale_scatter(None, idx, idx, w, nseg, mode="splat")
  return p[:, :, 0]


def _sc_multi_segsum(parts):
  mult = NW * CH
  idx_cat, w_cat, metas = [], [], []
  row_off = 0
  for idx, w, nseg in parts:
    nsegp = _cdiv(nseg, 128) * 128
    idx_cat.append(_pad1(idx, mult, fill=nseg) + row_off)
    w_cat.append(_pad1(w, mult))
    metas.append((row_off, nseg))
    row_off += nsegp
  p = _sc_gather_scale_scatter(
      None, jnp.concatenate(idx_cat), jnp.concatenate(idx_cat),
      jnp.concatenate(w_cat), row_off, mode="splat", seg_padded=True)
  return [p[:, o:o + n, 0] for (o, n) in metas]


def _sc_multi_agg(parts):
  mult = NW * CH
  tab_cat, src_cat, dst_cat, w_cat, metas = [], [], [], [], []
  row_off = 0
  tab_off = 0
  for y, src, dst, w, nseg in parts:
    nsegp = _cdiv(nseg, 128) * 128
    tab_cat.append(y)
    src_cat.append(_pad1(src, mult) + tab_off)
    dst_cat.append(_pad1(dst, mult, fill=nseg) + row_off)
    w_cat.append(_pad1(w, mult))
    metas.append((row_off, nseg))
    row_off += nsegp
    tab_off += y.shape[0]
  p = _sc_gather_scale_scatter(
      jnp.concatenate(tab_cat), jnp.concatenate(src_cat),
      jnp.concatenate(dst_cat), jnp.concatenate(w_cat), row_off,
      seg_padded=True, core_split=0.36)
  return [p[:, o:o + n, :] for (o, n) in metas]


def _sc_gather_scale_scatter(y, src, dst, w, nseg, mode="gather_scale",
                             seg_padded=False, core_split=None):
  ep = src.shape[0]
  assert ep % (NW * CH) == 0
  eper = ep // NW
  nch = eper // CH
  if core_split is not None:
    total_ch = ep // CH
    nch0 = max(1, int(round(total_ch * core_split / NS)))
    nch1 = total_ch // NS - nch0
  else:
    nch0 = nch1 = nch
  d = 128 if mode == "splat" else y.shape[1]
  nblk = d // LANES
  if seg_padded:
    nsegp = nseg
    assert nsegp % 128 == 0
  else:
    nsegp = _cdiv(nseg, 128) * 128
    assert nsegp > nseg
  rows_sub = nsegp // NS
  assert rows_sub % 8 == 0
  full = rows_sub // CH
  rem = rows_sub % CH

  mesh = plsc.VectorSubcoreMesh(core_axis_name="c", subcore_axis_name="s")
  has_gather = mode != "splat"

  scratch = [
      pltpu.VMEM((CH,), I32),
      pltpu.VMEM((CH,), I32),
      pltpu.VMEM((CH,), F32),
      pltpu.VMEM((CH, d), F32),
      pltpu.VMEM_SHARED((nsegp, d), F32),
      pltpu.SemaphoreType.DMA,
  ]

  @functools.partial(
      pl.kernel,
      mesh=mesh,
      out_type=jax.ShapeDtypeStruct((NC, nsegp, d), F32),
      scratch_types=scratch,
  )
  def k(*refs):
    if has_gather:
      (y_hbm, src_hbm, dst_hbm, w_hbm, out_hbm,
       src_v, dst_v, w_v, rows, acc, sem) = refs
    else:
      (src_hbm, dst_hbm, w_hbm, out_hbm,
       src_v, dst_v, w_v, rows, acc, sem) = refs
    c = lax.axis_index("c")
    s = lax.axis_index("s")
    z16 = jnp.zeros((LANES,), F32)

    def zrow(i, _):
      for jb in range(nblk):
        rows[i, pl.ds(jb * LANES, LANES)] = z16
      return 0

    lax.fori_loop(0, CH, zrow, 0)

    row0 = s * rows_sub

    def zacc(kk, _):
      o = pl.multiple_of(row0 + kk * CH, 8)
      pltpu.sync_copy(rows, acc.at[pl.ds(o, CH)])
      return 0

    lax.fori_loop(0, full, zacc, 0)
    if rem:
      pltpu.sync_copy(rows.at[pl.ds(0, rem)],
                      acc.at[pl.ds(row0 + full * CH, rem)])
    plsc.subcore_barrier()

    nch_me = jnp.where(c == 0, nch0, nch1)
    base = jnp.where(c == 0, s * nch0 * CH, (NS * nch0 + s * nch1) * CH)

    def ch_body(t, _):
      off = pl.multiple_of(base + t * CH, CH)
      if has_gather:
        pltpu.sync_copy(src_hbm.at[pl.ds(off, CH)], src_v)
        pltpu.async_copy(y_hbm.at[src_v], rows, sem).wait()
      if mode != "gather":
        pltpu.sync_copy(w_hbm.at[pl.ds(off, CH)], w_v)
      pltpu.sync_copy(dst_hbm.at[pl.ds(off, CH)], dst_v)

      def scale(g, _):
        gbase = pl.multiple_of(g * LANES, LANES)
        wv = w_v[pl.ds(gbase, LANES)]
        for j in range(LANES):
          ws = wv[j]
          e = gbase + j
          for jb in range(nblk):
            if mode == "splat":
              rows[e, pl.ds(jb * LANES, LANES)] = jnp.ones((LANES,), F32) * ws
            else:
              rows[e, pl.ds(jb * LANES, LANES)] = (
                  rows[e, pl.ds(jb * LANES, LANES)] * ws)
        return 0

      if mode != "gather":
        lax.fori_loop(0, CH // LANES, scale, 0)
      pltpu.sync_copy(rows, acc.at[dst_v], add=True)
      return 0

    lax.fori_loop(0, nch_me, ch_body, 0)
    plsc.subcore_barrier()

    def export(kk, _):
      o = pl.multiple_of(row0 + kk * CH, 8)
      pltpu.sync_copy(acc.at[pl.ds(o, CH)], out_hbm.at[c, pl.ds(o, CH)])
      return 0

    lax.fori_loop(0, full, export, 0)
    if rem:
      o = row0 + full * CH
      pltpu.sync_copy(acc.at[pl.ds(o, rem)], out_hbm.at[c, pl.ds(o, rem)])

  if has_gather:
    res = k(y, src, dst, w)
  else:
    res = k(src, dst, w)
  if nsegp != nseg:
    res = res[:, :nseg, :]
  return res


def _mm(x, W, bias=None, rowscale=None, colscale=None, act=None,
        bm=None, bn=None, bk=None):
  M, K = x.shape
  dout = W.shape[0]
  if bm is None:
    bm = M if M <= 2048 else 1000
  bn = bn or dout
  bk = bk or K
  nm, nn, nk = _cdiv(M, bm), _cdiv(dout, bn), _cdiv(K, bk)

  has_b = bias is not None
  has_rs = rowscale is not None
  has_cs = colscale is not None

  in_specs = [
      pl.BlockSpec((bm, bk), lambda m, n, k: (m, k)),
      pl.BlockSpec((bn, bk), lambda m, n, k: (n, k)),
  ]
  args = [x, W]
  if has_b:
    in_specs.append(pl.BlockSpec((1, bn), lambda m, n, k: (0, n)))
    args.append(bias.reshape(1, -1))
  if has_rs:
    in_specs.append(pl.BlockSpec((bm, 1), lambda m, n, k: (m, 0)))
    args.append(rowscale.reshape(-1, 1))
  if has_cs:
    in_specs.append(pl.BlockSpec((1, bn), lambda m, n, k: (0, n)))
    args.append(colscale.reshape(1, -1))

  def body(*refs):
    x_ref, w_ref = refs[0], refs[1]
    rest = list(refs[2:-2])
    o_ref, acc = refs[-2], refs[-1]
    kk = pl.program_id(2)

    @pl.when(kk == 0)
    def _():
      acc[...] = jnp.zeros_like(acc)

    acc[...] += lax.dot_general(
        x_ref[...], w_ref[...], (((1,), (1,)), ((), ())),
        preferred_element_type=F32)

    @pl.when(kk == nk - 1)
    def _():
      r = acc[...]
      i = 0
      if has_b:
        b_ref = rest[i]
        i += 1
      if has_rs:
        rs_ref = rest[i]
        i += 1
      if has_cs:
        cs_ref = rest[i]
        i += 1
      if has_cs:
        r = r * cs_ref[...]
      if has_b:
        r = r + b_ref[...]
      if has_rs:
        r = r * rs_ref[...]
      if act == "relu":
        r = jnp.maximum(r, 0.0)
      elif act == "leaky":
        r = jnp.where(r > 0, r, 0.01 * r)
      o_ref[...] = r

  return pl.pallas_call(
      body,
      grid=(nm, nn, nk),
      in_specs=in_specs,
      out_specs=pl.BlockSpec((bm, bn), lambda m, n, k: (m, n)),
      out_shape=jax.ShapeDtypeStruct((M, dout), F32),
      scratch_shapes=[pltpu.VMEM((bm, bn), F32)],
      compiler_params=pltpu.CompilerParams(
          dimension_semantics=("parallel", "parallel", "arbitrary")),
  )(*args)


def _attn(q, k, v, lvalid, want_w):
  BH, Lp, hd = q.shape
  scale = 1.0 / np.sqrt(hd)

  def body(q_ref, k_ref, v_ref, o_ref, *maybe_w):
    qq = q_ref[0]
    kk = k_ref[0]
    vv = v_ref[0]
    s = lax.dot_general(qq, kk, (((1,), (1,)), ((), ())),
                        preferred_element_type=F32) * scale
    colid = lax.broadcasted_iota(I32, (Lp, Lp), 1)
    s = jnp.where(colid < lvalid, s, -1e30)
    m = jnp.max(s, axis=1, keepdims=True)
    e = jnp.exp(s - m)
    den = jnp.sum(e, axis=1, keepdims=True)
    wgt = e / den
    o_ref[0] = lax.dot_general(wgt, vv, (((1,), (0,)), ((), ())),
                               preferred_element_type=F32)
    if want_w:
      maybe_w[0][0] = wgt

  spec3 = pl.BlockSpec((1, Lp, hd), lambda b: (b, 0, 0))
  out_shapes = [jax.ShapeDtypeStruct((BH, Lp, hd), F32)]
  out_specs = [spec3]
  if want_w:
    out_shapes.append(jax.ShapeDtypeStruct((BH, Lp, Lp), F32))
    out_specs.append(pl.BlockSpec((1, Lp, Lp), lambda b: (b, 0, 0)))

  res = pl.pallas_call(
      body,
      grid=(BH,),
      in_specs=[spec3, spec3, spec3],
      out_specs=out_specs,
      out_shape=out_shapes,
      compiler_params=pltpu.CompilerParams(
          dimension_semantics=("parallel",)),
  )(q, k, v)
  if want_w:
    return res[0], res[1]
  return res[0], None


def _ln_res(x, a, g, b):
  M, D = x.shape
  bm = M if M <= 2048 else 1000

  def body(x_ref, a_ref, g_ref, b_ref, o_ref):
    r = x_ref[...] + a_ref[...]
    m = jnp.mean(r, axis=1, keepdims=True)
    var = jnp.mean((r - m) ** 2, axis=1, keepdims=True)
    o_ref[...] = (r - m) / jnp.sqrt(var + 1e-5) * g_ref[...] + b_ref[...]

  return pl.pallas_call(
      body,
      grid=(_cdiv(M, bm),),
      in_specs=[
          pl.BlockSpec((bm, D), lambda m: (m, 0)),
          pl.BlockSpec((bm, D), lambda m: (m, 0)),
          pl.BlockSpec((1, D), lambda m: (0, 0)),
          pl.BlockSpec((1, D), lambda m: (0, 0)),
      ],
      out_specs=pl.BlockSpec((bm, D), lambda m: (m, 0)),
      out_shape=jax.ShapeDtypeStruct((M, D), F32),
  )(x, a, g.reshape(1, -1), b.reshape(1, -1))


def _ew_dinv(degp):
  npart, nseg = degp.shape
  bn = nseg if nseg <= 2048 else 2048

  def body(d_ref, o_ref):
    sdeg = 1.0 + jnp.sum(d_ref[...], axis=0)
    o_ref[...] = (1.0 / jnp.sqrt(jnp.maximum(sdeg, 1e-12)))[:, None]

  return pl.pallas_call(
      body,
      grid=(_cdiv(nseg, bn),),
      in_specs=[pl.BlockSpec((npart, bn), lambda n: (0, n))],
      out_specs=pl.BlockSpec((bn, 1), lambda n: (n, 0)),
      out_shape=jax.ShapeDtypeStruct((nseg, 1), F32),
  )(degp)


def _ew_gcn_update(p0, p1, y, dinv, b):
  M, D = y.shape
  bm = M if M <= 2048 else 1000

  def body(p0_ref, p1_ref, y_ref, s_ref, b_ref, o_ref):
    r = (p0_ref[...] + p1_ref[...] + y_ref[...]) * s_ref[...] + b_ref[...]
    o_ref[...] = jnp.maximum(r, 0.0)

  return pl.pallas_call(
      body,
      grid=(_cdiv(M, bm),),
      in_specs=[
          pl.BlockSpec((bm, D), lambda m: (m, 0)),
          pl.BlockSpec((bm, D), lambda m: (m, 0)),
          pl.BlockSpec((bm, D), lambda m: (m, 0)),
          pl.BlockSpec((bm, 1), lambda m: (m, 0)),
          pl.BlockSpec((1, D), lambda m: (0, 0)),
      ],
      out_specs=pl.BlockSpec((bm, D), lambda m: (m, 0)),
      out_shape=jax.ShapeDtypeStruct((M, D), F32),
  )(p0, p1, y, dinv, b.reshape(1, -1))


def _ew_pool(s0, s1, cntp, roi):
  M, D = s0.shape
  npart = cntp.shape[0]

  def body(s0_ref, s1_ref, c_ref, r_ref, p_ref, e_ref):
    cnt = jnp.sum(c_ref[...], axis=0)[:, None]
    ssum = s0_ref[...] + s1_ref[...]
    pooled = jnp.where(cnt > 0, ssum / jnp.maximum(cnt, 1.0), 0.0)
    p_ref[...] = pooled
    e_ref[...] = pooled + r_ref[...]

  return pl.pallas_call(
      body,
      grid=(1,),
      in_specs=[
          pl.BlockSpec((M, D), lambda m: (0, 0)),
          pl.BlockSpec((M, D), lambda m: (0, 0)),
          pl.BlockSpec((npart, M), lambda m: (0, 0)),
          pl.BlockSpec((M, D), lambda m: (0, 0)),
      ],
      out_specs=[
          pl.BlockSpec((M, D), lambda m: (0, 0)),
          pl.BlockSpec((M, D), lambda m: (0, 0)),
      ],
      out_shape=[
          jax.ShapeDtypeStruct((M, D), F32),
          jax.ShapeDtypeStruct((M, D), F32),
      ],
  )(s0, s1, cntp, roi)


def _pad1(a, mult, fill=0):
  n = a.shape[0]
  npad = _cdiv(n, mult) * mult - n
  if npad == 0:
    return a
  return jnp.concatenate([a, jnp.full((npad,), fill, a.dtype)])


def _gcn_stack(x0, src, dst, w, nseg, layers):
  mult = NW * CH
  srcp = _pad1(src, mult)
  dstp = _pad1(dst, mult, fill=nseg)
  wp = _pad1(w, mult)
  degp = _sc_segsum_scalar(dstp, wp, nseg)
  dinv = _ew_dinv(degp)
  h = x0
  for (W, b) in layers:
    y = _mm(h, W, rowscale=dinv)
    p = _sc_gather_scale_scatter(y, srcp, dstp, wp, nseg)
    h = _ew_gcn_update(p[0], p[1], y, dinv, b)
  return h


def _attn_block(xin, p, bc, L, Lp, want_w):
  heads = 4
  d = xin.shape[1]
  hd = d // heads
  q = _mm(xin, p["q"][0], bias=p["q"][1])
  kmat = _mm(xin, p["k"][0], bias=p["k"][1])
  v = _mm(xin, p["v"][0], bias=p["v"][1])

  def to_heads(t):
    tp = t.reshape(bc, L, d)
    if Lp != L:
      tp = jnp.pad(tp, ((0, 0), (0, Lp - L), (0, 0)))
    return tp.reshape(bc, Lp, heads, hd).transpose(0, 2, 1, 3).reshape(
        bc * heads, Lp, hd)

  a, wgt = _attn(to_heads(q), to_heads(kmat), to_heads(v), L, want_w)
  a = a.reshape(bc, heads, Lp, hd).transpose(0, 2, 1, 3).reshape(bc, Lp, d)
  a = a[:, :L].reshape(bc * L, d)
  o = _mm(a, p["o"][0], bias=p["o"][1])
  x1 = _ln_res(xin, o, p["ln1"][0], p["ln1"][1])
  f1 = _mm(x1, p["ff1"][0], bias=p["ff1"][1], act="relu")
  f2 = _mm(f1, p["ff2"][0], bias=p["ff2"][1])
  x2 = _ln_res(x1, f2, p["ln2"][0], p["ln2"][1])
  if want_w:
    wgt = wgt.reshape(bc, heads, Lp, Lp)[:, :, :L, :L]
  return x2, wgt


def kernel(x, edge_index, edge_attr, batch, roi_x, roi_edge_index,
           roi_edge_attr, batch2, params):
  N = 10000
  B = 8
  R = 148
  N2 = B * R

  feats = x[:, :128].astype(F32)
  node_label = x[:, 128].astype(I32)
  x2 = roi_x[:, :128].astype(F32)
  src = edge_index[0].astype(I32)
  dst = edge_index[1].astype(I32)
  ew = edge_attr.astype(F32)
  rs = roi_edge_index[0].astype(I32)
  rd = roi_edge_index[1].astype(I32)
  rew = roi_edge_attr.astype(F32)

  poolidx = batch.astype(I32) * R + node_label
  ones_n = jnp.ones((N,), F32)
  degp_roi, degp_main, cntp = _sc_multi_segsum(
      [(rd, rew, N2), (dst, ew, N), (poolidx, ones_n, N2)])
  dinv_roi = _ew_dinv(degp_roi)
  dinv_main = _ew_dinv(degp_main)

  h_roi, h_main = x2, feats
  for (wr, br), (wm, bm) in zip(params["gcn_roi"], params["gcn"]):
    y_roi = _mm(h_roi, wr, rowscale=dinv_roi)
    y_main = _mm(h_main, wm, rowscale=dinv_main)
    p_roi, p_main = _sc_multi_agg(
        [(y_roi, rs, rd, rew, N2), (y_main, src, dst, ew, N)])
    h_roi = _ew_gcn_update(p_roi[0], p_roi[1], y_roi, dinv_roi, br)
    h_main = _ew_gcn_update(p_main[0], p_main[1], y_main, dinv_main, bm)
  x2f = h_roi
  h = h_main
  embedding_roi = x2f.reshape(B, R, 128)

  upd, _ = _attn_block(h, params["mha"], B, 1250, 1280, False)
  updated_embeddings = upd

  mult = NW * CH
  idxp = _pad1(poolidx, mult, fill=N2)
  onesp = _pad1(ones_n, mult)
  srcp = _pad1(jnp.arange(N, dtype=I32), mult)
  sump = _sc_gather_scale_scatter(h, srcp, idxp, onesp, N2, mode="gather")
  emb_flat, emb_sum_flat = _ew_pool(sump[0], sump[1], cntp, x2f)
  embedding = emb_flat.reshape(B, R, 128)
  emb_sum = emb_sum_flat.reshape(B, R, 128)

  t_flat, attn_w = _attn_block(emb_sum_flat, params["attn_sum"], B, R, 160,
                               True)
  t_out = t_flat.reshape(B, R, 128)

  flat = t_flat.reshape(B, R * 128)
  g, bb, rm, rv = params["bn"]
  A = g / jnp.sqrt(rv + 1e-5)
  C = (params["clf1"][1] - rm) * A + bb
  z = _mm(flat, params["clf1"][0], bias=C, colscale=A, act="leaky",
          bm=8, bn=1000, bk=512)
  out = _mm(z, params["clf2"][0], bias=params["clf2"][1], bm=8)

  return (out, embedding, embedding_roi, emb_sum, t_out, attn_w,
          updated_embeddings)


# speedup vs baseline: 1.0915x; 1.0915x over previous
import functools

import jax
import jax.numpy as jnp
import numpy as np
from jax import lax
from jax.experimental import pallas as pl
from jax.experimental.pallas import tpu as pltpu
from jax.experimental.pallas import tpu_sc as plsc

F32 = jnp.float32
I32 = jnp.int32

NC = 2
NS = 16
NW = NC * NS
LANES = 16
CH = 128


def _cdiv(a, b):
  return (a + b - 1) // b


def _sc_segsum_scalar(idx, w, nseg):
  p = _sc_gather_scale_scatter(None, idx, idx, w, nseg, mode="splat")
  return p[:, :, 0]


def _sc_multi_segsum(parts):
  mult = NW * CH
  idx_cat, w_cat, metas = [], [], []
  row_off = 0
  for idx, w, nseg in parts:
    nsegp = _cdiv(nseg, 128) * 128
    idx_cat.append(_pad1(idx, mult, fill=nseg) + row_off)
    w_cat.append(_pad1(w, mult))
    metas.append((row_off, nseg))
    row_off += nsegp
  p = _sc_gather_scale_scatter(
      None, jnp.concatenate(idx_cat), jnp.concatenate(idx_cat),
      jnp.concatenate(w_cat), row_off, mode="splat", seg_padded=True)
  return [p[:, o:o + n, 0] for (o, n) in metas]


def _sc_multi_agg(parts):
  mult = NW * CH
  tab_cat, src_cat, dst_cat, w_cat, metas = [], [], [], [], []
  row_off = 0
  tab_off = 0
  for y, src, dst, w, nseg in parts:
    nsegp = _cdiv(nseg, 128) * 128
    tab_cat.append(y)
    src_cat.append(_pad1(src, mult) + tab_off)
    dst_cat.append(_pad1(dst, mult, fill=nseg) + row_off)
    w_cat.append(_pad1(w, mult))
    metas.append((row_off, nseg))
    row_off += nsegp
    tab_off += y.shape[0]
  p = _sc_gather_scale_scatter(
      jnp.concatenate(tab_cat), jnp.concatenate(src_cat),
      jnp.concatenate(dst_cat), jnp.concatenate(w_cat), row_off,
      seg_padded=True, core_split=0.64)
  return [p[:, o:o + n, :] for (o, n) in metas]


def _sc_gather_scale_scatter(y, src, dst, w, nseg, mode="gather_scale",
                             seg_padded=False, core_split=None):
  ep = src.shape[0]
  assert ep % (NW * CH) == 0
  eper = ep // NW
  nch = eper // CH
  if core_split is not None:
    total_ch = ep // CH
    nch0 = max(1, int(round(total_ch * core_split / NS)))
    nch1 = total_ch // NS - nch0
  else:
    nch0 = nch1 = nch
  d = 128 if mode == "splat" else y.shape[1]
  nblk = d // LANES
  if seg_padded:
    nsegp = nseg
    assert nsegp % 128 == 0
  else:
    nsegp = _cdiv(nseg, 128) * 128
    assert nsegp > nseg
  rows_sub = nsegp // NS
  assert rows_sub % 8 == 0
  full = rows_sub // CH
  rem = rows_sub % CH

  mesh = plsc.VectorSubcoreMesh(core_axis_name="c", subcore_axis_name="s")
  has_gather = mode != "splat"

  scratch = [
      pltpu.VMEM((CH,), I32),
      pltpu.VMEM((CH,), I32),
      pltpu.VMEM((CH,), F32),
      pltpu.VMEM((CH, d), F32),
      pltpu.VMEM_SHARED((nsegp, d), F32),
      pltpu.SemaphoreType.DMA,
  ]

  @functools.partial(
      pl.kernel,
      mesh=mesh,
      out_type=jax.ShapeDtypeStruct((NC, nsegp, d), F32),
      scratch_types=scratch,
  )
  def k(*refs):
    if has_gather:
      (y_hbm, src_hbm, dst_hbm, w_hbm, out_hbm,
       src_v, dst_v, w_v, rows, acc, sem) = refs
    else:
      (src_hbm, dst_hbm, w_hbm, out_hbm,
       src_v, dst_v, w_v, rows, acc, sem) = refs
    c = lax.axis_index("c")
    s = lax.axis_index("s")
    z16 = jnp.zeros((LANES,), F32)

    def zrow(i, _):
      for jb in range(nblk):
        rows[i, pl.ds(jb * LANES, LANES)] = z16
      return 0

    lax.fori_loop(0, CH, zrow, 0)

    row0 = s * rows_sub

    def zacc(kk, _):
      o = pl.multiple_of(row0 + kk * CH, 8)
      pltpu.sync_copy(rows, acc.at[pl.ds(o, CH)])
      return 0

    lax.fori_loop(0, full, zacc, 0)
    if rem:
      pltpu.sync_copy(rows.at[pl.ds(0, rem)],
                      acc.at[pl.ds(row0 + full * CH, rem)])
    plsc.subcore_barrier()

    nch_me = jnp.where(c == 0, nch0, nch1)
    base = jnp.where(c == 0, s * nch0 * CH, (NS * nch0 + s * nch1) * CH)

    def ch_body(t, _):
      off = pl.multiple_of(base + t * CH, CH)
      if has_gather:
        pltpu.sync_copy(src_hbm.at[pl.ds(off, CH)], src_v)
        pltpu.async_copy(y_hbm.at[src_v], rows, sem).wait()
      if mode != "gather":
        pltpu.sync_copy(w_hbm.at[pl.ds(off, CH)], w_v)
      pltpu.sync_copy(dst_hbm.at[pl.ds(off, CH)], dst_v)

      def scale(g, _):
        gbase = pl.multiple_of(g * LANES, LANES)
        wv = w_v[pl.ds(gbase, LANES)]
        for j in range(LANES):
          ws = wv[j]
          e = gbase + j
          for jb in range(nblk):
            if mode == "splat":
              rows[e, pl.ds(jb * LANES, LANES)] = jnp.ones((LANES,), F32) * ws
            else:
              rows[e, pl.ds(jb * LANES, LANES)] = (
                  rows[e, pl.ds(jb * LANES, LANES)] * ws)
        return 0

      if mode != "gather":
        lax.fori_loop(0, CH // LANES, scale, 0)
      pltpu.sync_copy(rows, acc.at[dst_v], add=True)
      return 0

    lax.fori_loop(0, nch_me, ch_body, 0)
    plsc.subcore_barrier()

    def export(kk, _):
      o = pl.multiple_of(row0 + kk * CH, 8)
      pltpu.sync_copy(acc.at[pl.ds(o, CH)], out_hbm.at[c, pl.ds(o, CH)])
      return 0

    lax.fori_loop(0, full, export, 0)
    if rem:
      o = row0 + full * CH
      pltpu.sync_copy(acc.at[pl.ds(o, rem)], out_hbm.at[c, pl.ds(o, rem)])

  if has_gather:
    res = k(y, src, dst, w)
  else:
    res = k(src, dst, w)
  if nsegp != nseg:
    res = res[:, :nseg, :]
  return res


def _mm(x, W, bias=None, rowscale=None, colscale=None, act=None,
        bm=None, bn=None, bk=None):
  M, K = x.shape
  dout = W.shape[0]
  if bm is None:
    bm = M if M <= 2048 else 1000
  bn = bn or dout
  bk = bk or K
  nm, nn, nk = _cdiv(M, bm), _cdiv(dout, bn), _cdiv(K, bk)

  has_b = bias is not None
  has_rs = rowscale is not None
  has_cs = colscale is not None

  in_specs = [
      pl.BlockSpec((bm, bk), lambda m, n, k: (m, k)),
      pl.BlockSpec((bn, bk), lambda m, n, k: (n, k)),
  ]
  args = [x, W]
  if has_b:
    in_specs.append(pl.BlockSpec((1, bn), lambda m, n, k: (0, n)))
    args.append(bias.reshape(1, -1))
  if has_rs:
    in_specs.append(pl.BlockSpec((bm, 1), lambda m, n, k: (m, 0)))
    args.append(rowscale.reshape(-1, 1))
  if has_cs:
    in_specs.append(pl.BlockSpec((1, bn), lambda m, n, k: (0, n)))
    args.append(colscale.reshape(1, -1))

  def body(*refs):
    x_ref, w_ref = refs[0], refs[1]
    rest = list(refs[2:-2])
    o_ref, acc = refs[-2], refs[-1]
    kk = pl.program_id(2)

    @pl.when(kk == 0)
    def _():
      acc[...] = jnp.zeros_like(acc)

    acc[...] += lax.dot_general(
        x_ref[...], w_ref[...], (((1,), (1,)), ((), ())),
        preferred_element_type=F32)

    @pl.when(kk == nk - 1)
    def _():
      r = acc[...]
      i = 0
      if has_b:
        b_ref = rest[i]
        i += 1
      if has_rs:
        rs_ref = rest[i]
        i += 1
      if has_cs:
        cs_ref = rest[i]
        i += 1
      if has_cs:
        r = r * cs_ref[...]
      if has_b:
        r = r + b_ref[...]
      if has_rs:
        r = r * rs_ref[...]
      if act == "relu":
        r = jnp.maximum(r, 0.0)
      elif act == "leaky":
        r = jnp.where(r > 0, r, 0.01 * r)
      o_ref[...] = r

  return pl.pallas_call(
      body,
      grid=(nm, nn, nk),
      in_specs=in_specs,
      out_specs=pl.BlockSpec((bm, bn), lambda m, n, k: (m, n)),
      out_shape=jax.ShapeDtypeStruct((M, dout), F32),
      scratch_shapes=[pltpu.VMEM((bm, bn), F32)],
      compiler_params=pltpu.CompilerParams(
          dimension_semantics=("parallel", "parallel", "arbitrary")),
  )(*args)


def _attn(q, k, v, lvalid, want_w):
  BH, Lp, hd = q.shape
  scale = 1.0 / np.sqrt(hd)

  def body(q_ref, k_ref, v_ref, o_ref, *maybe_w):
    qq = q_ref[0]
    kk = k_ref[0]
    vv = v_ref[0]
    s = lax.dot_general(qq, kk, (((1,), (1,)), ((), ())),
                        preferred_element_type=F32) * scale
    colid = lax.broadcasted_iota(I32, (Lp, Lp), 1)
    s = jnp.where(colid < lvalid, s, -1e30)
    m = jnp.max(s, axis=1, keepdims=True)
    e = jnp.exp(s - m)
    den = jnp.sum(e, axis=1, keepdims=True)
    wgt = e / den
    o_ref[0] = lax.dot_general(wgt, vv, (((1,), (0,)), ((), ())),
                               preferred_element_type=F32)
    if want_w:
      maybe_w[0][0] = wgt

  spec3 = pl.BlockSpec((1, Lp, hd), lambda b: (b, 0, 0))
  out_shapes = [jax.ShapeDtypeStruct((BH, Lp, hd), F32)]
  out_specs = [spec3]
  if want_w:
    out_shapes.append(jax.ShapeDtypeStruct((BH, Lp, Lp), F32))
    out_specs.append(pl.BlockSpec((1, Lp, Lp), lambda b: (b, 0, 0)))

  res = pl.pallas_call(
      body,
      grid=(BH,),
      in_specs=[spec3, spec3, spec3],
      out_specs=out_specs,
      out_shape=out_shapes,
      compiler_params=pltpu.CompilerParams(
          dimension_semantics=("parallel",)),
  )(q, k, v)
  if want_w:
    return res[0], res[1]
  return res[0], None


def _ln_res(x, a, g, b):
  M, D = x.shape
  bm = M if M <= 2048 else 1000

  def body(x_ref, a_ref, g_ref, b_ref, o_ref):
    r = x_ref[...] + a_ref[...]
    m = jnp.mean(r, axis=1, keepdims=True)
    var = jnp.mean((r - m) ** 2, axis=1, keepdims=True)
    o_ref[...] = (r - m) / jnp.sqrt(var + 1e-5) * g_ref[...] + b_ref[...]

  return pl.pallas_call(
      body,
      grid=(_cdiv(M, bm),),
      in_specs=[
          pl.BlockSpec((bm, D), lambda m: (m, 0)),
          pl.BlockSpec((bm, D), lambda m: (m, 0)),
          pl.BlockSpec((1, D), lambda m: (0, 0)),
          pl.BlockSpec((1, D), lambda m: (0, 0)),
      ],
      out_specs=pl.BlockSpec((bm, D), lambda m: (m, 0)),
      out_shape=jax.ShapeDtypeStruct((M, D), F32),
  )(x, a, g.reshape(1, -1), b.reshape(1, -1))


def _ew_dinv(degp):
  npart, nseg = degp.shape
  bn = nseg if nseg <= 2048 else 2048

  def body(d_ref, o_ref):
    sdeg = 1.0 + jnp.sum(d_ref[...], axis=0)
    o_ref[...] = (1.0 / jnp.sqrt(jnp.maximum(sdeg, 1e-12)))[:, None]

  return pl.pallas_call(
      body,
      grid=(_cdiv(nseg, bn),),
      in_specs=[pl.BlockSpec((npart, bn), lambda n: (0, n))],
      out_specs=pl.BlockSpec((bn, 1), lambda n: (n, 0)),
      out_shape=jax.ShapeDtypeStruct((nseg, 1), F32),
  )(degp)


def _ew_gcn_update(p0, p1, y, dinv, b):
  M, D = y.shape
  bm = M if M <= 2048 else 1000

  def body(p0_ref, p1_ref, y_ref, s_ref, b_ref, o_ref):
    r = (p0_ref[...] + p1_ref[...] + y_ref[...]) * s_ref[...] + b_ref[...]
    o_ref[...] = jnp.maximum(r, 0.0)

  return pl.pallas_call(
      body,
      grid=(_cdiv(M, bm),),
      in_specs=[
          pl.BlockSpec((bm, D), lambda m: (m, 0)),
          pl.BlockSpec((bm, D), lambda m: (m, 0)),
          pl.BlockSpec((bm, D), lambda m: (m, 0)),
          pl.BlockSpec((bm, 1), lambda m: (m, 0)),
          pl.BlockSpec((1, D), lambda m: (0, 0)),
      ],
      out_specs=pl.BlockSpec((bm, D), lambda m: (m, 0)),
      out_shape=jax.ShapeDtypeStruct((M, D), F32),
  )(p0, p1, y, dinv, b.reshape(1, -1))


def _ew_pool(s0, s1, cntp, roi):
  M, D = s0.shape
  npart = cntp.shape[0]

  def body(s0_ref, s1_ref, c_ref, r_ref, p_ref, e_ref):
    cnt = jnp.sum(c_ref[...], axis=0)[:, None]
    ssum = s0_ref[...] + s1_ref[...]
    pooled = jnp.where(cnt > 0, ssum / jnp.maximum(cnt, 1.0), 0.0)
    p_ref[...] = pooled
    e_ref[...] = pooled + r_ref[...]

  return pl.pallas_call(
      body,
      grid=(1,),
      in_specs=[
          pl.BlockSpec((M, D), lambda m: (0, 0)),
          pl.BlockSpec((M, D), lambda m: (0, 0)),
          pl.BlockSpec((npart, M), lambda m: (0, 0)),
          pl.BlockSpec((M, D), lambda m: (0, 0)),
      ],
      out_specs=[
          pl.BlockSpec((M, D), lambda m: (0, 0)),
          pl.BlockSpec((M, D), lambda m: (0, 0)),
      ],
      out_shape=[
          jax.ShapeDtypeStruct((M, D), F32),
          jax.ShapeDtypeStruct((M, D), F32),
      ],
  )(s0, s1, cntp, roi)


def _pad1(a, mult, fill=0):
  n = a.shape[0]
  npad = _cdiv(n, mult) * mult - n
  if npad == 0:
    return a
  return jnp.concatenate([a, jnp.full((npad,), fill, a.dtype)])


def _gcn_stack(x0, src, dst, w, nseg, layers):
  mult = NW * CH
  srcp = _pad1(src, mult)
  dstp = _pad1(dst, mult, fill=nseg)
  wp = _pad1(w, mult)
  degp = _sc_segsum_scalar(dstp, wp, nseg)
  dinv = _ew_dinv(degp)
  h = x0
  for (W, b) in layers:
    y = _mm(h, W, rowscale=dinv)
    p = _sc_gather_scale_scatter(y, srcp, dstp, wp, nseg)
    h = _ew_gcn_update(p[0], p[1], y, dinv, b)
  return h


def _attn_block(xin, p, bc, L, Lp, want_w):
  heads = 4
  d = xin.shape[1]
  hd = d // heads
  q = _mm(xin, p["q"][0], bias=p["q"][1])
  kmat = _mm(xin, p["k"][0], bias=p["k"][1])
  v = _mm(xin, p["v"][0], bias=p["v"][1])

  def to_heads(t):
    tp = t.reshape(bc, L, d)
    if Lp != L:
      tp = jnp.pad(tp, ((0, 0), (0, Lp - L), (0, 0)))
    return tp.reshape(bc, Lp, heads, hd).transpose(0, 2, 1, 3).reshape(
        bc * heads, Lp, hd)

  a, wgt = _attn(to_heads(q), to_heads(kmat), to_heads(v), L, want_w)
  a = a.reshape(bc, heads, Lp, hd).transpose(0, 2, 1, 3).reshape(bc, Lp, d)
  a = a[:, :L].reshape(bc * L, d)
  o = _mm(a, p["o"][0], bias=p["o"][1])
  x1 = _ln_res(xin, o, p["ln1"][0], p["ln1"][1])
  f1 = _mm(x1, p["ff1"][0], bias=p["ff1"][1], act="relu")
  f2 = _mm(f1, p["ff2"][0], bias=p["ff2"][1])
  x2 = _ln_res(x1, f2, p["ln2"][0], p["ln2"][1])
  if want_w:
    wgt = wgt.reshape(bc, heads, Lp, Lp)[:, :, :L, :L]
  return x2, wgt


def kernel(x, edge_index, edge_attr, batch, roi_x, roi_edge_index,
           roi_edge_attr, batch2, params):
  N = 10000
  B = 8
  R = 148
  N2 = B * R

  feats = x[:, :128].astype(F32)
  node_label = x[:, 128].astype(I32)
  x2 = roi_x[:, :128].astype(F32)
  src = edge_index[0].astype(I32)
  dst = edge_index[1].astype(I32)
  ew = edge_attr.astype(F32)
  rs = roi_edge_index[0].astype(I32)
  rd = roi_edge_index[1].astype(I32)
  rew = roi_edge_attr.astype(F32)

  poolidx = batch.astype(I32) * R + node_label
  ones_n = jnp.ones((N,), F32)
  degp_roi, degp_main, cntp = _sc_multi_segsum(
      [(rd, rew, N2), (dst, ew, N), (poolidx, ones_n, N2)])
  dinv_roi = _ew_dinv(degp_roi)
  dinv_main = _ew_dinv(degp_main)

  h_roi, h_main = x2, feats
  for (wr, br), (wm, bm) in zip(params["gcn_roi"], params["gcn"]):
    y_roi = _mm(h_roi, wr, rowscale=dinv_roi)
    y_main = _mm(h_main, wm, rowscale=dinv_main)
    p_roi, p_main = _sc_multi_agg(
        [(y_roi, rs, rd, rew, N2), (y_main, src, dst, ew, N)])
    h_roi = _ew_gcn_update(p_roi[0], p_roi[1], y_roi, dinv_roi, br)
    h_main = _ew_gcn_update(p_main[0], p_main[1], y_main, dinv_main, bm)
  x2f = h_roi
  h = h_main
  embedding_roi = x2f.reshape(B, R, 128)

  upd, _ = _attn_block(h, params["mha"], B, 1250, 1280, False)
  updated_embeddings = upd

  mult = NW * CH
  idxp = _pad1(poolidx, mult, fill=N2)
  onesp = _pad1(ones_n, mult)
  srcp = _pad1(jnp.arange(N, dtype=I32), mult)
  sump = _sc_gather_scale_scatter(h, srcp, idxp, onesp, N2, mode="gather")
  emb_flat, emb_sum_flat = _ew_pool(sump[0], sump[1], cntp, x2f)
  embedding = emb_flat.reshape(B, R, 128)
  emb_sum = emb_sum_flat.reshape(B, R, 128)

  t_flat, attn_w = _attn_block(emb_sum_flat, params["attn_sum"], B, R, 160,
                               True)
  t_out = t_flat.reshape(B, R, 128)

  flat = t_flat.reshape(B, R * 128)
  g, bb, rm, rv = params["bn"]
  A = g / jnp.sqrt(rv + 1e-5)
  C = (params["clf1"][1] - rm) * A + bb
  z = _mm(flat, params["clf1"][0], bias=C, colscale=A, act="leaky",
          bm=8, bn=1000, bk=512)
  out = _mm(z, params["clf2"][0], bias=params["clf2"][1], bm=8)

  return (out, embedding, embedding_roi, emb_sum, t_out, attn_w,
          updated_embeddings)


# speedup vs baseline: 1.0997x; 1.0075x over previous
import functools

import jax
import jax.numpy as jnp
import numpy as np
from jax import lax
from jax.experimental import pallas as pl
from jax.experimental.pallas import tpu as pltpu
from jax.experimental.pallas import tpu_sc as plsc

F32 = jnp.float32
I32 = jnp.int32

NC = 2
NS = 16
NW = NC * NS
LANES = 16
CH = 128


def _cdiv(a, b):
  return (a + b - 1) // b


def _sc_segsum_scalar(idx, w, nseg):
  p = _sc_gather_scale_scatter(None, idx, idx, w, nseg, mode="splat")
  return p[:, :, 0]


def _sc_multi_segsum(parts):
  mult = NW * CH
  idx_cat, w_cat, metas = [], [], []
  row_off = 0
  for idx, w, nseg in parts:
    nsegp = _cdiv(nseg, 128) * 128
    idx_cat.append(_pad1(idx, mult, fill=nseg) + row_off)
    w_cat.append(_pad1(w, mult))
    metas.append((row_off, nseg))
    row_off += nsegp
  p = _sc_gather_scale_scatter(
      None, jnp.concatenate(idx_cat), jnp.concatenate(idx_cat),
      jnp.concatenate(w_cat), row_off, mode="splat", seg_padded=True)
  return [p[:, o:o + n, 0] for (o, n) in metas]


def _sc_multi_agg(parts):
  mult = NW * CH
  tab_cat, src_cat, dst_cat, w_cat, metas = [], [], [], [], []
  row_off = 0
  tab_off = 0
  for y, src, dst, w, nseg in parts:
    nsegp = _cdiv(nseg, 128) * 128
    tab_cat.append(y)
    src_cat.append(_pad1(src, mult) + tab_off)
    dst_cat.append(_pad1(dst, mult, fill=nseg) + row_off)
    w_cat.append(_pad1(w, mult))
    metas.append((row_off, nseg))
    row_off += nsegp
    tab_off += y.shape[0]
  p = _sc_gather_scale_scatter(
      jnp.concatenate(tab_cat), jnp.concatenate(src_cat),
      jnp.concatenate(dst_cat), jnp.concatenate(w_cat), row_off,
      seg_padded=True, core_split=0.58)
  return [p[:, o:o + n, :] for (o, n) in metas]


def _sc_gather_scale_scatter(y, src, dst, w, nseg, mode="gather_scale",
                             seg_padded=False, core_split=None):
  ep = src.shape[0]
  assert ep % (NW * CH) == 0
  eper = ep // NW
  nch = eper // CH
  if core_split is not None:
    total_ch = ep // CH
    nch0 = max(1, int(round(total_ch * core_split / NS)))
    nch1 = total_ch // NS - nch0
  else:
    nch0 = nch1 = nch
  d = 128 if mode == "splat" else y.shape[1]
  nblk = d // LANES
  if seg_padded:
    nsegp = nseg
    assert nsegp % 128 == 0
  else:
    nsegp = _cdiv(nseg, 128) * 128
    assert nsegp > nseg
  rows_sub = nsegp // NS
  assert rows_sub % 8 == 0
  full = rows_sub // CH
  rem = rows_sub % CH

  mesh = plsc.VectorSubcoreMesh(core_axis_name="c", subcore_axis_name="s")
  has_gather = mode != "splat"

  scratch = [
      pltpu.VMEM((CH,), I32),
      pltpu.VMEM((CH,), I32),
      pltpu.VMEM((CH,), F32),
      pltpu.VMEM((CH, d), F32),
      pltpu.VMEM_SHARED((nsegp, d), F32),
      pltpu.SemaphoreType.DMA,
  ]

  @functools.partial(
      pl.kernel,
      mesh=mesh,
      out_type=jax.ShapeDtypeStruct((NC, nsegp, d), F32),
      scratch_types=scratch,
  )
  def k(*refs):
    if has_gather:
      (y_hbm, src_hbm, dst_hbm, w_hbm, out_hbm,
       src_v, dst_v, w_v, rows, acc, sem) = refs
    else:
      (src_hbm, dst_hbm, w_hbm, out_hbm,
       src_v, dst_v, w_v, rows, acc, sem) = refs
    c = lax.axis_index("c")
    s = lax.axis_index("s")
    z16 = jnp.zeros((LANES,), F32)

    def zrow(i, _):
      for jb in range(nblk):
        rows[i, pl.ds(jb * LANES, LANES)] = z16
      return 0

    lax.fori_loop(0, CH, zrow, 0)

    row0 = s * rows_sub

    def zacc(kk, _):
      o = pl.multiple_of(row0 + kk * CH, 8)
      pltpu.sync_copy(rows, acc.at[pl.ds(o, CH)])
      return 0

    lax.fori_loop(0, full, zacc, 0)
    if rem:
      pltpu.sync_copy(rows.at[pl.ds(0, rem)],
                      acc.at[pl.ds(row0 + full * CH, rem)])
    plsc.subcore_barrier()

    nch_me = jnp.where(c == 0, nch0, nch1)
    base = jnp.where(c == 0, s * nch0 * CH, (NS * nch0 + s * nch1) * CH)

    def ch_body(t, _):
      off = pl.multiple_of(base + t * CH, CH)
      if has_gather:
        pltpu.sync_copy(src_hbm.at[pl.ds(off, CH)], src_v)
        pltpu.async_copy(y_hbm.at[src_v], rows, sem).wait()
      if mode != "gather":
        pltpu.sync_copy(w_hbm.at[pl.ds(off, CH)], w_v)
      pltpu.sync_copy(dst_hbm.at[pl.ds(off, CH)], dst_v)

      def scale(g, _):
        gbase = pl.multiple_of(g * LANES, LANES)
        wv = w_v[pl.ds(gbase, LANES)]
        for j in range(LANES):
          ws = wv[j]
          e = gbase + j
          for jb in range(nblk):
            if mode == "splat":
              rows[e, pl.ds(jb * LANES, LANES)] = jnp.ones((LANES,), F32) * ws
            else:
              rows[e, pl.ds(jb * LANES, LANES)] = (
                  rows[e, pl.ds(jb * LANES, LANES)] * ws)
        return 0

      if mode != "gather":
        lax.fori_loop(0, CH // LANES, scale, 0)
      pltpu.sync_copy(rows, acc.at[dst_v], add=True)
      return 0

    lax.fori_loop(0, nch_me, ch_body, 0)
    plsc.subcore_barrier()

    def export(kk, _):
      o = pl.multiple_of(row0 + kk * CH, 8)
      pltpu.sync_copy(acc.at[pl.ds(o, CH)], out_hbm.at[c, pl.ds(o, CH)])
      return 0

    lax.fori_loop(0, full, export, 0)
    if rem:
      o = row0 + full * CH
      pltpu.sync_copy(acc.at[pl.ds(o, rem)], out_hbm.at[c, pl.ds(o, rem)])

  if has_gather:
    res = k(y, src, dst, w)
  else:
    res = k(src, dst, w)
  if nsegp != nseg:
    res = res[:, :nseg, :]
  return res


def _mm(x, W, bias=None, rowscale=None, colscale=None, act=None,
        bm=None, bn=None, bk=None):
  M, K = x.shape
  dout = W.shape[0]
  if bm is None:
    bm = M if M <= 2048 else 1000
  bn = bn or dout
  bk = bk or K
  nm, nn, nk = _cdiv(M, bm), _cdiv(dout, bn), _cdiv(K, bk)

  has_b = bias is not None
  has_rs = rowscale is not None
  has_cs = colscale is not None

  in_specs = [
      pl.BlockSpec((bm, bk), lambda m, n, k: (m, k)),
      pl.BlockSpec((bn, bk), lambda m, n, k: (n, k)),
  ]
  args = [x, W]
  if has_b:
    in_specs.append(pl.BlockSpec((1, bn), lambda m, n, k: (0, n)))
    args.append(bias.reshape(1, -1))
  if has_rs:
    in_specs.append(pl.BlockSpec((bm, 1), lambda m, n, k: (m, 0)))
    args.append(rowscale.reshape(-1, 1))
  if has_cs:
    in_specs.append(pl.BlockSpec((1, bn), lambda m, n, k: (0, n)))
    args.append(colscale.reshape(1, -1))

  def body(*refs):
    x_ref, w_ref = refs[0], refs[1]
    rest = list(refs[2:-2])
    o_ref, acc = refs[-2], refs[-1]
    kk = pl.program_id(2)

    @pl.when(kk == 0)
    def _():
      acc[...] = jnp.zeros_like(acc)

    acc[...] += lax.dot_general(
        x_ref[...], w_ref[...], (((1,), (1,)), ((), ())),
        preferred_element_type=F32)

    @pl.when(kk == nk - 1)
    def _():
      r = acc[...]
      i = 0
      if has_b:
        b_ref = rest[i]
        i += 1
      if has_rs:
        rs_ref = rest[i]
        i += 1
      if has_cs:
        cs_ref = rest[i]
        i += 1
      if has_cs:
        r = r * cs_ref[...]
      if has_b:
        r = r + b_ref[...]
      if has_rs:
        r = r * rs_ref[...]
      if act == "relu":
        r = jnp.maximum(r, 0.0)
      elif act == "leaky":
        r = jnp.where(r > 0, r, 0.01 * r)
      o_ref[...] = r

  return pl.pallas_call(
      body,
      grid=(nm, nn, nk),
      in_specs=in_specs,
      out_specs=pl.BlockSpec((bm, bn), lambda m, n, k: (m, n)),
      out_shape=jax.ShapeDtypeStruct((M, dout), F32),
      scratch_shapes=[pltpu.VMEM((bm, bn), F32)],
      compiler_params=pltpu.CompilerParams(
          dimension_semantics=("parallel", "parallel", "arbitrary")),
  )(*args)


def _attn(q, k, v, lvalid, want_w):
  BH, Lp, hd = q.shape
  scale = 1.0 / np.sqrt(hd)

  def body(q_ref, k_ref, v_ref, o_ref, *maybe_w):
    qq = q_ref[0]
    kk = k_ref[0]
    vv = v_ref[0]
    s = lax.dot_general(qq, kk, (((1,), (1,)), ((), ())),
                        preferred_element_type=F32) * scale
    colid = lax.broadcasted_iota(I32, (Lp, Lp), 1)
    s = jnp.where(colid < lvalid, s, -1e30)
    m = jnp.max(s, axis=1, keepdims=True)
    e = jnp.exp(s - m)
    den = jnp.sum(e, axis=1, keepdims=True)
    wgt = e / den
    o_ref[0] = lax.dot_general(wgt, vv, (((1,), (0,)), ((), ())),
                               preferred_element_type=F32)
    if want_w:
      maybe_w[0][0] = wgt

  spec3 = pl.BlockSpec((1, Lp, hd), lambda b: (b, 0, 0))
  out_shapes = [jax.ShapeDtypeStruct((BH, Lp, hd), F32)]
  out_specs = [spec3]
  if want_w:
    out_shapes.append(jax.ShapeDtypeStruct((BH, Lp, Lp), F32))
    out_specs.append(pl.BlockSpec((1, Lp, Lp), lambda b: (b, 0, 0)))

  res = pl.pallas_call(
      body,
      grid=(BH,),
      in_specs=[spec3, spec3, spec3],
      out_specs=out_specs,
      out_shape=out_shapes,
      compiler_params=pltpu.CompilerParams(
          dimension_semantics=("parallel",)),
  )(q, k, v)
  if want_w:
    return res[0], res[1]
  return res[0], None


def _ln_res(x, a, g, b):
  M, D = x.shape
  bm = M if M <= 2048 else 1000

  def body(x_ref, a_ref, g_ref, b_ref, o_ref):
    r = x_ref[...] + a_ref[...]
    m = jnp.mean(r, axis=1, keepdims=True)
    var = jnp.mean((r - m) ** 2, axis=1, keepdims=True)
    o_ref[...] = (r - m) / jnp.sqrt(var + 1e-5) * g_ref[...] + b_ref[...]

  return pl.pallas_call(
      body,
      grid=(_cdiv(M, bm),),
      in_specs=[
          pl.BlockSpec((bm, D), lambda m: (m, 0)),
          pl.BlockSpec((bm, D), lambda m: (m, 0)),
          pl.BlockSpec((1, D), lambda m: (0, 0)),
          pl.BlockSpec((1, D), lambda m: (0, 0)),
      ],
      out_specs=pl.BlockSpec((bm, D), lambda m: (m, 0)),
      out_shape=jax.ShapeDtypeStruct((M, D), F32),
  )(x, a, g.reshape(1, -1), b.reshape(1, -1))


def _ew_dinv(degp):
  npart, nseg = degp.shape
  bn = nseg if nseg <= 2048 else 2048

  def body(d_ref, o_ref):
    sdeg = 1.0 + jnp.sum(d_ref[...], axis=0)
    o_ref[...] = (1.0 / jnp.sqrt(jnp.maximum(sdeg, 1e-12)))[:, None]

  return pl.pallas_call(
      body,
      grid=(_cdiv(nseg, bn),),
      in_specs=[pl.BlockSpec((npart, bn), lambda n: (0, n))],
      out_specs=pl.BlockSpec((bn, 1), lambda n: (n, 0)),
      out_shape=jax.ShapeDtypeStruct((nseg, 1), F32),
  )(degp)


def _ew_gcn_update(p0, p1, y, dinv, b):
  M, D = y.shape
  bm = M if M <= 2048 else 1000

  def body(p0_ref, p1_ref, y_ref, s_ref, b_ref, o_ref):
    r = (p0_ref[...] + p1_ref[...] + y_ref[...]) * s_ref[...] + b_ref[...]
    o_ref[...] = jnp.maximum(r, 0.0)

  return pl.pallas_call(
      body,
      grid=(_cdiv(M, bm),),
      in_specs=[
          pl.BlockSpec((bm, D), lambda m: (m, 0)),
          pl.BlockSpec((bm, D), lambda m: (m, 0)),
          pl.BlockSpec((bm, D), lambda m: (m, 0)),
          pl.BlockSpec((bm, 1), lambda m: (m, 0)),
          pl.BlockSpec((1, D), lambda m: (0, 0)),
      ],
      out_specs=pl.BlockSpec((bm, D), lambda m: (m, 0)),
      out_shape=jax.ShapeDtypeStruct((M, D), F32),
  )(p0, p1, y, dinv, b.reshape(1, -1))


def _ew_pool(s0, s1, cntp, roi):
  M, D = s0.shape
  npart = cntp.shape[0]

  def body(s0_ref, s1_ref, c_ref, r_ref, p_ref, e_ref):
    cnt = jnp.sum(c_ref[...], axis=0)[:, None]
    ssum = s0_ref[...] + s1_ref[...]
    pooled = jnp.where(cnt > 0, ssum / jnp.maximum(cnt, 1.0), 0.0)
    p_ref[...] = pooled
    e_ref[...] = pooled + r_ref[...]

  return pl.pallas_call(
      body,
      grid=(1,),
      in_specs=[
          pl.BlockSpec((M, D), lambda m: (0, 0)),
          pl.BlockSpec((M, D), lambda m: (0, 0)),
          pl.BlockSpec((npart, M), lambda m: (0, 0)),
          pl.BlockSpec((M, D), lambda m: (0, 0)),
      ],
      out_specs=[
          pl.BlockSpec((M, D), lambda m: (0, 0)),
          pl.BlockSpec((M, D), lambda m: (0, 0)),
      ],
      out_shape=[
          jax.ShapeDtypeStruct((M, D), F32),
          jax.ShapeDtypeStruct((M, D), F32),
      ],
  )(s0, s1, cntp, roi)


def _pad1(a, mult, fill=0):
  n = a.shape[0]
  npad = _cdiv(n, mult) * mult - n
  if npad == 0:
    return a
  return jnp.concatenate([a, jnp.full((npad,), fill, a.dtype)])


def _gcn_stack(x0, src, dst, w, nseg, layers):
  mult = NW * CH
  srcp = _pad1(src, mult)
  dstp = _pad1(dst, mult, fill=nseg)
  wp = _pad1(w, mult)
  degp = _sc_segsum_scalar(dstp, wp, nseg)
  dinv = _ew_dinv(degp)
  h = x0
  for (W, b) in layers:
    y = _mm(h, W, rowscale=dinv)
    p = _sc_gather_scale_scatter(y, srcp, dstp, wp, nseg)
    h = _ew_gcn_update(p[0], p[1], y, dinv, b)
  return h


def _attn_block(xin, p, bc, L, Lp, want_w):
  heads = 4
  d = xin.shape[1]
  hd = d // heads
  q = _mm(xin, p["q"][0], bias=p["q"][1])
  kmat = _mm(xin, p["k"][0], bias=p["k"][1])
  v = _mm(xin, p["v"][0], bias=p["v"][1])

  def to_heads(t):
    tp = t.reshape(bc, L, d)
    if Lp != L:
      tp = jnp.pad(tp, ((0, 0), (0, Lp - L), (0, 0)))
    return tp.reshape(bc, Lp, heads, hd).transpose(0, 2, 1, 3).reshape(
        bc * heads, Lp, hd)

  a, wgt = _attn(to_heads(q), to_heads(kmat), to_heads(v), L, want_w)
  a = a.reshape(bc, heads, Lp, hd).transpose(0, 2, 1, 3).reshape(bc, Lp, d)
  a = a[:, :L].reshape(bc * L, d)
  o = _mm(a, p["o"][0], bias=p["o"][1])
  x1 = _ln_res(xin, o, p["ln1"][0], p["ln1"][1])
  f1 = _mm(x1, p["ff1"][0], bias=p["ff1"][1], act="relu")
  f2 = _mm(f1, p["ff2"][0], bias=p["ff2"][1])
  x2 = _ln_res(x1, f2, p["ln2"][0], p["ln2"][1])
  if want_w:
    wgt = wgt.reshape(bc, heads, Lp, Lp)[:, :, :L, :L]
  return x2, wgt


def kernel(x, edge_index, edge_attr, batch, roi_x, roi_edge_index,
           roi_edge_attr, batch2, params):
  N = 10000
  B = 8
  R = 148
  N2 = B * R

  feats = x[:, :128].astype(F32)
  node_label = x[:, 128].astype(I32)
  x2 = roi_x[:, :128].astype(F32)
  src = edge_index[0].astype(I32)
  dst = edge_index[1].astype(I32)
  ew = edge_attr.astype(F32)
  rs = roi_edge_index[0].astype(I32)
  rd = roi_edge_index[1].astype(I32)
  rew = roi_edge_attr.astype(F32)

  poolidx = batch.astype(I32) * R + node_label
  ones_n = jnp.ones((N,), F32)
  degp_roi, degp_main, cntp = _sc_multi_segsum(
      [(rd, rew, N2), (dst, ew, N), (poolidx, ones_n, N2)])
  dinv_roi = _ew_dinv(degp_roi)
  dinv_main = _ew_dinv(degp_main)

  h_roi, h_main = x2, feats
  for (wr, br), (wm, bm) in zip(params["gcn_roi"], params["gcn"]):
    y_roi = _mm(h_roi, wr, rowscale=dinv_roi)
    y_main = _mm(h_main, wm, rowscale=dinv_main)
    p_roi, p_main = _sc_multi_agg(
        [(y_roi, rs, rd, rew, N2), (y_main, src, dst, ew, N)])
    h_roi = _ew_gcn_update(p_roi[0], p_roi[1], y_roi, dinv_roi, br)
    h_main = _ew_gcn_update(p_main[0], p_main[1], y_main, dinv_main, bm)
  x2f = h_roi
  h = h_main
  embedding_roi = x2f.reshape(B, R, 128)

  upd, _ = _attn_block(h, params["mha"], B, 1250, 1280, False)
  updated_embeddings = upd

  mult = NW * CH
  idxp = _pad1(poolidx, mult, fill=N2)
  onesp = _pad1(ones_n, mult)
  srcp = _pad1(jnp.arange(N, dtype=I32), mult)
  sump = _sc_gather_scale_scatter(h, srcp, idxp, onesp, N2, mode="gather")
  emb_flat, emb_sum_flat = _ew_pool(sump[0], sump[1], cntp, x2f)
  embedding = emb_flat.reshape(B, R, 128)
  emb_sum = emb_sum_flat.reshape(B, R, 128)

  t_flat, attn_w = _attn_block(emb_sum_flat, params["attn_sum"], B, R, 160,
                               True)
  t_out = t_flat.reshape(B, R, 128)

  flat = t_flat.reshape(B, R * 128)
  g, bb, rm, rv = params["bn"]
  A = g / jnp.sqrt(rv + 1e-5)
  C = (params["clf1"][1] - rm) * A + bb
  z = _mm(flat, params["clf1"][0], bias=C, colscale=A, act="leaky",
          bm=8, bn=1000, bk=512)
  out = _mm(z, params["clf2"][0], bias=params["clf2"][1], bm=8)

  return (out, embedding, embedding_roi, emb_sum, t_out, attn_w,
          updated_embeddings)
